# baseline (speedup 1.0000x reference)
import functools

import jax
import jax.numpy as jnp
from jax import lax
from jax.experimental import pallas as pl
from jax.experimental.pallas import tpu as pltpu
from jax.experimental.pallas import tpu_sc as plsc

B = 16384
EMB = 16
HID = 32
PACK = 8
PW = PACK * EMB
NC = 2
NS = 16
NW = NC * NS
BPW = B // NW
CH = 128
NCH = BPW // CH
VREG = 16


def _sc_gather(user, quiz, utab_p, qtab_p):
    mesh = plsc.VectorSubcoreMesh(core_axis_name="c", subcore_axis_name="s")

    @functools.partial(
        pl.kernel,
        mesh=mesh,
        out_type=[
            jax.ShapeDtypeStruct((B, PW), jnp.float32),
            jax.ShapeDtypeStruct((B, PW), jnp.float32),
        ],
        scratch_types=[
            pltpu.VMEM((NCH, CH), jnp.int32),
            pltpu.VMEM((NCH, CH), jnp.int32),
            pltpu.VMEM((2, CH, PW), jnp.float32),
            pltpu.VMEM((2, CH, PW), jnp.float32),
            pltpu.SemaphoreType.DMA,
            pltpu.SemaphoreType.DMA,
        ],
    )
    def k(user_hbm, quiz_hbm, utab_hbm, qtab_hbm, uout_hbm, qout_hbm,
          uidx_v, qidx_v, ubuf, qbuf, usem, qsem):
        wid = lax.axis_index("s") * NC + lax.axis_index("c")
        base = wid * BPW
        for j in range(NCH):
            pltpu.sync_copy(user_hbm.at[pl.ds(base + j * CH, CH)], uidx_v.at[j])
            pltpu.sync_copy(quiz_hbm.at[pl.ds(base + j * CH, CH)], qidx_v.at[j])
        for j in range(NCH):
            for v in range(CH // VREG):
                s = pl.ds(v * VREG, VREG)
                uidx_v[j, s] = uidx_v[j, s] >> 3
                qidx_v[j, s] = qidx_v[j, s] >> 3

        def gstart(j):
            return (
                pltpu.async_copy(utab_hbm.at[uidx_v.at[j]], ubuf.at[j % 2], usem),
                pltpu.async_copy(qtab_hbm.at[qidx_v.at[j]], qbuf.at[j % 2], qsem),
            )

        gc = {0: gstart(0)}
        for j in range(NCH):
            if j + 1 < NCH:
                gc[j + 1] = gstart(j + 1)
            uc, qc = gc[j]
            uc.wait()
            qc.wait()
            pltpu.sync_copy(ubuf.at[j % 2], uout_hbm.at[pl.ds(base + j * CH, CH)])
            pltpu.sync_copy(qbuf.at[j % 2], qout_hbm.at[pl.ds(base + j * CH, CH)])

    return k(user, quiz, utab_p, qtab_p)


def _mlp_body(up_ref, qp_ref, uid_ref, qid_ref, t_ref, w1u_ref, w1q_ref,
              w1t_ref, b1_ref, w2_ref, b2_ref, o_ref):
    lane = lax.broadcasted_iota(jnp.int32, (up_ref.shape[0], PW), 1)
    grp = lane >> 4
    um = jnp.where(grp == (uid_ref[...] & 7), up_ref[...], 0.0)
    qm = jnp.where(grp == (qid_ref[...] & 7), qp_ref[...], 0.0)
    x = (jnp.dot(um, w1u_ref[...], preferred_element_type=jnp.float32)
         + jnp.dot(qm, w1q_ref[...], preferred_element_type=jnp.float32)
         + t_ref[...] * w1t_ref[...]
         + b1_ref[...])
    h = jnp.maximum(x, 0.0)
    z = jnp.dot(h, w2_ref[...], preferred_element_type=jnp.float32) + b2_ref[...]
    o_ref[...] = 1.0 / (1.0 + jnp.exp(-z))


def _mlp(up, qp, uid, qid, time, W1, b1, W2, b2):
    RB = 2048
    grid = (B // RB,)
    W1u = jnp.tile(W1[:EMB], (PACK, 1))
    W1q = jnp.tile(W1[EMB:2 * EMB], (PACK, 1))
    w1t = W1[2 * EMB:]
    out = pl.pallas_call(
        _mlp_body,
        grid=grid,
        in_specs=[
            pl.BlockSpec((RB, PW), lambda i: (i, 0)),
            pl.BlockSpec((RB, PW), lambda i: (i, 0)),
            pl.BlockSpec((RB, 1), lambda i: (i, 0)),
            pl.BlockSpec((RB, 1), lambda i: (i, 0)),
            pl.BlockSpec((RB, 1), lambda i: (i, 0)),
            pl.BlockSpec((PW, HID), lambda i: (0, 0)),
            pl.BlockSpec((PW, HID), lambda i: (0, 0)),
            pl.BlockSpec((1, HID), lambda i: (0, 0)),
            pl.BlockSpec((1, HID), lambda i: (0, 0)),
            pl.BlockSpec((HID, 1), lambda i: (0, 0)),
            pl.BlockSpec((1, 1), lambda i: (0, 0)),
        ],
        out_specs=pl.BlockSpec((RB, 1), lambda i: (i, 0)),
        out_shape=jax.ShapeDtypeStruct((B, 1), jnp.float32),
    )(up, qp, uid, qid, time, W1u, W1q, w1t, b1.reshape(1, HID), W2,
      b2.reshape(1, 1))
    return out.reshape(B)


def kernel(user, quiz, time, user_table, quiz_table, W1, b1, W2, b2):
    uid = user.astype(jnp.int32)
    qid = quiz.astype(jnp.int32)
    utab_p = user_table.reshape(-1, PW)
    qtab_p = quiz_table.reshape(-1, PW)
    up, qp = _sc_gather(uid, qid, utab_p, qtab_p)
    return _mlp(up, qp, uid.reshape(B, 1), qid.reshape(B, 1), time,
                W1, b1, W2, b2)
